# Initial kernel scaffold; baseline (speedup 1.0000x reference)
import jax
import jax.numpy as jnp
from jax.experimental import pallas as pl


def kernel(x, edge_index, edge_attr, batch, global_attr, params):
    raise NotImplementedError("write your pallas kernel here")


# speedup vs baseline: 1.0507x; 1.0507x over previous
import functools

import jax
import jax.numpy as jnp
from jax import lax
from jax.experimental import pallas as pl

N = 100000
E = 1600000
B = 8
HID = 64
HEADS = 4
C = 16


def _layer_norm(x, g, b):
    mu = x.mean(-1, keepdims=True)
    var = x.var(-1, keepdims=True)
    return (x - mu) / jnp.sqrt(var + 1e-5) * g + b


def _enc_body(x_ref, w_ref, b_ref, g_ref, be_ref, o_ref):
    h = jnp.dot(x_ref[...], w_ref[...], preferred_element_type=jnp.float32)
    h = h + b_ref[...][None, :]
    mu = jnp.mean(h, axis=-1, keepdims=True)
    var = jnp.mean((h - mu) * (h - mu), axis=-1, keepdims=True)
    h = (h - mu) * lax.rsqrt(var + 1e-5) * g_ref[...][None, :] + be_ref[...][None, :]
    o_ref[...] = jnp.maximum(h, 0.0)


def _encoder_pallas(x, p, block_rows):
    n, din = x.shape
    dout = p["W"].shape[1]
    grid = n // block_rows
    return pl.pallas_call(
        _enc_body,
        grid=(grid,),
        in_specs=[
            pl.BlockSpec((block_rows, din), lambda i: (i, 0)),
            pl.BlockSpec((din, dout), lambda i: (0, 0)),
            pl.BlockSpec((dout,), lambda i: (0,)),
            pl.BlockSpec((dout,), lambda i: (0,)),
            pl.BlockSpec((dout,), lambda i: (0,)),
        ],
        out_specs=pl.BlockSpec((block_rows, dout), lambda i: (i, 0)),
        out_shape=jax.ShapeDtypeStruct((n, dout), jnp.float32),
    )(x, p["W"], p["b"], p["g"], p["be"])


def _gatv2(x, src, dst, e_attr, loop_attr, p):
    loop_idx = jnp.arange(N, dtype=src.dtype)
    src_f = jnp.concatenate([src, loop_idx])
    dst_f = jnp.concatenate([dst, loop_idx])
    ea = jnp.concatenate([e_attr, loop_attr], axis=0)
    x_l = (x @ p["W_l"] + p["b_l"]).reshape(N, HEADS, C)
    x_r = (x @ p["W_r"] + p["b_r"]).reshape(N, HEADS, C)
    e = (ea @ p["W_e"]).reshape(-1, HEADS, C)
    m = x_l[src_f] + x_r[dst_f] + e
    m = jax.nn.leaky_relu(m, 0.2)
    alpha = (m * p["att"][None]).sum(-1)
    ex = jnp.exp(alpha)
    denom = jax.ops.segment_sum(ex, dst_f, num_segments=N)
    out = jax.ops.segment_sum(x_l[src_f] * ex[..., None], dst_f, num_segments=N)
    out = out / (denom[..., None] + 1e-16)
    return out.reshape(N, HID) + p["bias"]


def kernel(x, edge_index, edge_attr, batch, global_attr, params):
    src, dst = edge_index[0], edge_index[1]
    h = _encoder_pallas(x, params["node_enc"], 2000)
    ea = _encoder_pallas(edge_attr, params["edge_enc"], 8000)
    g = _layer_norm(global_attr @ params["glob_enc"]["W"] + params["glob_enc"]["b"],
                    params["glob_enc"]["g"], params["glob_enc"]["be"])
    g = jax.nn.relu(g)

    deg = jax.ops.segment_sum(jnp.ones(E, jnp.float32), dst, num_segments=N)
    sum_e = jax.ops.segment_sum(ea, dst, num_segments=N)
    loop_attr = sum_e / jnp.clip(deg, 1.0)[:, None]

    for cp, nrm in zip(params["convs"], params["norms"]):
        h_res = h
        h = _gatv2(h, src, dst, ea, loop_attr, cp)
        h = _layer_norm(h, nrm["g"], nrm["be"])
        h = jax.nn.leaky_relu(h, 0.2)
        h = h + h_res

    cnt = jax.ops.segment_sum(jnp.ones(N, jnp.float32), batch, num_segments=B)
    x_mean = jax.ops.segment_sum(h, batch, num_segments=B) / jnp.clip(cnt, 1.0)[:, None]
    x_max = jax.ops.segment_max(h, batch, num_segments=B)
    fused = jnp.concatenate([x_mean, x_max, g], axis=-1)
    hh = jax.nn.relu(fused @ params["head"]["W1"] + params["head"]["b1"])
    out = hh @ params["head"]["W2"] + params["head"]["b2"]
    return jnp.exp(out)


# speedup vs baseline: 52.4450x; 49.9126x over previous
import functools

import jax
import jax.numpy as jnp
from jax import lax
from jax.experimental import pallas as pl
from jax.experimental.pallas import tpu as pltpu
from jax.experimental.pallas import tpu_sc as plsc

N = 100000
E = 1600000
B = 8
HID = 64
HEADS = 4
C = 16

NBKT = 14
NBSZ = 8192
NPAD = NBKT * NBSZ
CH = 128
NCHUNK = E // CH
W128 = 128
BLK = 2048
EBLK = 8000
E4 = E // 4
E4BLK = 2000
ROWS_PER_TILE = NBSZ // 16

_mesh = plsc.VectorSubcoreMesh(core_axis_name="c", subcore_axis_name="s")


def _enc_body(x_ref, w_ref, b_ref, g_ref, be_ref, o_ref):
    h = jnp.dot(x_ref[...], w_ref[...], preferred_element_type=jnp.float32)
    h = h + b_ref[...][None, :]
    mu = jnp.mean(h, axis=-1, keepdims=True)
    var = jnp.mean((h - mu) * (h - mu), axis=-1, keepdims=True)
    h = (h - mu) * lax.rsqrt(var + 1e-5) * g_ref[...][None, :] + be_ref[...][None, :]
    h = jnp.maximum(h, 0.0)
    dout = h.shape[1]
    if dout < o_ref.shape[1]:
        h = jnp.concatenate(
            [h, jnp.zeros((h.shape[0], o_ref.shape[1] - dout), jnp.float32)], axis=-1)
    o_ref[...] = h


def _encoder_pallas(x, p, block_rows, out_w=None):
    n, din = x.shape
    dout = p["W"].shape[1]
    ow = dout if out_w is None else out_w
    return pl.pallas_call(
        _enc_body,
        grid=(n // block_rows,),
        in_specs=[
            pl.BlockSpec((block_rows, din), lambda i: (i, 0)),
            pl.BlockSpec((din, dout), lambda i: (0, 0)),
            pl.BlockSpec((dout,), lambda i: (0,)),
            pl.BlockSpec((dout,), lambda i: (0,)),
            pl.BlockSpec((dout,), lambda i: (0,)),
        ],
        out_specs=pl.BlockSpec((block_rows, ow), lambda i: (i, 0)),
        out_shape=jax.ShapeDtypeStruct((n, ow), jnp.float32),
    )(x, p["W"], p["b"], p["g"], p["be"])


def _pre_body(h_ref, ds_ref, wl_ref, bl_ref, wr_ref, br_ref, we_ref, att_ref,
              xlr_ref, acc_ref):
    hb = h_ref[...]
    xl = jnp.dot(hb, wl_ref[...], preferred_element_type=jnp.float32) + bl_ref[...][None, :]
    xr = jnp.dot(hb, wr_ref[...], preferred_element_type=jnp.float32) + br_ref[...][None, :]
    dsb = ds_ref[...]
    la = dsb[:, :32] / jnp.maximum(dsb[:, 32:33], 1.0)
    el = jnp.dot(la, we_ref[...], preferred_element_type=jnp.float32)
    m0 = xl + xr + el
    m0 = jnp.where(m0 >= 0, m0, 0.2 * m0)
    att = att_ref[...]
    parts = []
    exs = []
    for h4 in range(HEADS):
        sl = slice(16 * h4, 16 * h4 + 16)
        a0 = jnp.sum(m0[:, sl] * att[None, sl], axis=-1)
        e0 = jnp.exp(a0)
        exs.append(e0[:, None])
        parts.append(e0[:, None] * xl[:, sl])
    zpad = jnp.zeros((hb.shape[0], W128 - HID - HEADS), jnp.float32)
    acc_ref[...] = jnp.concatenate(parts + exs + [zpad], axis=-1)
    xlr_ref[...] = jnp.concatenate([xl, xr], axis=-1)


def _pre_pallas(h, degsum, cp, att_flat):
    grid = NPAD // BLK
    return pl.pallas_call(
        _pre_body,
        grid=(grid,),
        in_specs=[
            pl.BlockSpec((BLK, HID), lambda i: (i, 0)),
            pl.BlockSpec((BLK, W128), lambda i: (i, 0)),
            pl.BlockSpec((HID, HID), lambda i: (0, 0)),
            pl.BlockSpec((HID,), lambda i: (0,)),
            pl.BlockSpec((HID, HID), lambda i: (0, 0)),
            pl.BlockSpec((HID,), lambda i: (0,)),
            pl.BlockSpec((32, HID), lambda i: (0, 0)),
            pl.BlockSpec((HID,), lambda i: (0,)),
        ],
        out_specs=[
            pl.BlockSpec((BLK, W128), lambda i: (i, 0)),
            pl.BlockSpec((BLK, W128), lambda i: (i, 0)),
        ],
        out_shape=[
            jax.ShapeDtypeStruct((NPAD, W128), jnp.float32),
            jax.ShapeDtypeStruct((NPAD, W128), jnp.float32),
        ],
    )(h, degsum, cp["W_l"], cp["b_l"], cp["W_r"], cp["b_r"], cp["W_e"], att_flat)


def _ee_body(ea_ref, we_ref, o_ref):
    ea4 = ea_ref[...]
    outs = []
    for j in range(4):
        outs.append(jnp.dot(ea4[:, 32 * j:32 * j + 32], we_ref[...],
                            preferred_element_type=jnp.float32))
    o_ref[...] = jnp.concatenate(outs, axis=-1)


def _ee_pallas(ea_s4, we):
    return pl.pallas_call(
        _ee_body,
        grid=(E4 // E4BLK,),
        in_specs=[
            pl.BlockSpec((E4BLK, W128), lambda i: (i, 0)),
            pl.BlockSpec((32, HID), lambda i: (0, 0)),
        ],
        out_specs=pl.BlockSpec((E4BLK, 256), lambda i: (i, 0)),
        out_shape=jax.ShapeDtypeStruct((E4, 256), jnp.float32),
    )(ea_s4, we)


def _post_body(acc_ref, bias_ref, g_ref, be_ref, hres_ref, o_ref):
    a = acc_ref[...]
    outs = []
    for h4 in range(HEADS):
        outs.append(a[:, 16 * h4:16 * h4 + 16] / (a[:, 64 + h4:65 + h4] + 1e-16))
    o = jnp.concatenate(outs, axis=-1) + bias_ref[...][None, :]
    mu = jnp.mean(o, axis=-1, keepdims=True)
    var = jnp.mean((o - mu) * (o - mu), axis=-1, keepdims=True)
    o = (o - mu) * lax.rsqrt(var + 1e-5) * g_ref[...][None, :] + be_ref[...][None, :]
    o = jnp.where(o >= 0, o, 0.2 * o)
    o_ref[...] = o + hres_ref[...]


def _post_pallas(acc, bias, nrm, h_res):
    return pl.pallas_call(
        _post_body,
        grid=(NPAD // BLK,),
        in_specs=[
            pl.BlockSpec((BLK, W128), lambda i: (i, 0)),
            pl.BlockSpec((HID,), lambda i: (0,)),
            pl.BlockSpec((HID,), lambda i: (0,)),
            pl.BlockSpec((HID,), lambda i: (0,)),
            pl.BlockSpec((BLK, HID), lambda i: (i, 0)),
        ],
        out_specs=pl.BlockSpec((BLK, HID), lambda i: (i, 0)),
        out_shape=jax.ShapeDtypeStruct((NPAD, HID), jnp.float32),
    )(acc, bias, nrm["g"], nrm["be"], h_res)


def _pool_body(h_ref, b_ref, sum_ref, max_ref):
    hb = h_ref[...]
    bbf = b_ref[...]
    gid = lax.broadcasted_iota(jnp.int32, (hb.shape[0], B), 1).astype(jnp.float32)
    oh = (bbf == gid).astype(jnp.float32)
    s = lax.dot_general(oh, hb, (((0,), (0,)), ((), ())),
                        preferred_element_type=jnp.float32)
    sum_ref[...] = s[None]
    ms = []
    for bg in range(B):
        msk = bbf == float(bg)
        ms.append(jnp.max(jnp.where(msk, hb, -jnp.inf), axis=0, keepdims=True))
    max_ref[...] = jnp.concatenate(ms, axis=0)[None]


def _pool_pallas(h, batch_pad):
    grid = NPAD // BLK
    return pl.pallas_call(
        _pool_body,
        grid=(grid,),
        in_specs=[
            pl.BlockSpec((BLK, HID), lambda i: (i, 0)),
            pl.BlockSpec((BLK, 1), lambda i: (i, 0)),
        ],
        out_specs=[
            pl.BlockSpec((1, B, HID), lambda i: (i, 0, 0)),
            pl.BlockSpec((1, B, HID), lambda i: (i, 0, 0)),
        ],
        out_shape=[
            jax.ShapeDtypeStruct((grid, B, HID), jnp.float32),
            jax.ShapeDtypeStruct((grid, B, HID), jnp.float32),
        ],
    )(h, batch_pad)


def _head_body(sum_ref, max_ref, cnt_ref, ga_ref, wg_ref, bg_ref, gg_ref,
               beg_ref, w1_ref, b1_ref, w2_ref, b2_ref, o_ref):
    xm = sum_ref[...] / jnp.maximum(cnt_ref[...], 1.0)[:, None]
    xx = max_ref[...]
    ge = jnp.dot(ga_ref[...], wg_ref[...], preferred_element_type=jnp.float32)
    ge = ge + bg_ref[...][None, :]
    mu = jnp.mean(ge, axis=-1, keepdims=True)
    var = jnp.mean((ge - mu) * (ge - mu), axis=-1, keepdims=True)
    ge = (ge - mu) * lax.rsqrt(var + 1e-5) * gg_ref[...][None, :] + beg_ref[...][None, :]
    ge = jnp.maximum(ge, 0.0)
    w1 = w1_ref[...]
    hh = (jnp.dot(xm, w1[0:64], preferred_element_type=jnp.float32)
          + jnp.dot(xx, w1[64:128], preferred_element_type=jnp.float32)
          + jnp.dot(ge, w1[128:160], preferred_element_type=jnp.float32)
          + b1_ref[...][None, :])
    hh = jnp.maximum(hh, 0.0)
    o = jnp.sum(hh * w2_ref[...][None, :], axis=-1) + b2_ref[...]
    o_ref[...] = jnp.exp(o)


def _head_pallas(sums, maxs, cnt, ga, gp, hp):
    return pl.pallas_call(
        _head_body,
        out_shape=jax.ShapeDtypeStruct((B,), jnp.float32),
    )(sums, maxs, cnt, ga, gp["W"], gp["b"], gp["g"], gp["be"],
      hp["W1"], hp["b1"], hp["W2"].reshape(HID), jnp.broadcast_to(hp["b2"], (B,)))


def _vsum16(v, iota):
    for k in (8, 4, 2, 1):
        idx = jnp.bitwise_xor(iota, k)
        v = v + v.at[idx].get(mode="promise_in_bounds")
    return v


def _splat(v, j):
    idx = jnp.full((16,), j, jnp.int32)
    return v.at[idx].get(mode="promise_in_bounds")


@functools.partial(
    pl.kernel,
    out_type=(
        jax.ShapeDtypeStruct((E4, W128), jnp.float32),
        jax.ShapeDtypeStruct((NPAD, W128), jnp.float32),
    ),
    mesh=_mesh,
    scratch_types=[
        pltpu.VMEM_SHARED((NBSZ + 8, W128), jnp.float32),
        pltpu.VMEM((CH,), jnp.int32),
        pltpu.VMEM((CH,), jnp.int32),
        pltpu.VMEM((CH, W128), jnp.float32),
        pltpu.VMEM((CH // 4, W128), jnp.float32),
        pltpu.VMEM((CH, W128), jnp.float32),
        pltpu.VMEM((CH,), jnp.int32),
        pltpu.VMEM((16, 16), jnp.int32),
        pltpu.SemaphoreType.DMA,
    ],
)
def _k0(dsts_h, order_h, eap_h, zrows_h, bnds_h,
        eas4_out, degsum_out,
        spacc, ordb, dstb, eab, easb, contrib, lidxb, bndb, sem1):
    c = lax.axis_index("c")
    s = lax.axis_index("s")
    pltpu.sync_copy(bnds_h, bndb)
    iota = lax.iota(jnp.int32, 16)
    onehot32 = jnp.where(iota == 0, 1.0, 0.0)
    zeros16 = jnp.zeros((16,), jnp.float32)

    def zbody(e, carry):
        for j in range(5):
            contrib[e, pl.ds(48 + 16 * j, 16)] = zeros16
        return carry
    lax.fori_loop(0, CH, zbody, 0)

    for bi in range(NBKT // 2):
        bkt = 2 * bi + c
        base = bkt * NBSZ
        bv = bndb[bkt]
        lo = bv[0]
        hi = bv[1]
        pltpu.sync_copy(zrows_h.at[pl.ds(pl.multiple_of(s * ROWS_PER_TILE, ROWS_PER_TILE), ROWS_PER_TILE)],
                        spacc.at[pl.ds(pl.multiple_of(s * ROWS_PER_TILE, ROWS_PER_TILE), ROWS_PER_TILE)])
        plsc.subcore_barrier()
        nch = hi - lo
        nmine = jnp.maximum(0, (nch - s + 15) // 16)

        def chunk_body(i, carry):
            k = lo + s + 16 * i
            e0 = pl.multiple_of(k * CH, CH)
            pltpu.sync_copy(order_h.at[pl.ds(e0, CH)], ordb)
            pltpu.sync_copy(dsts_h.at[pl.ds(e0, CH)], dstb)
            cp1 = pltpu.async_copy(eap_h.at[ordb], eab, sem1)
            for g in range(CH // 16):
                dv = dstb[pl.ds(g * 16, 16)]
                lv = dv - base
                ok = (lv >= 0) & (lv < NBSZ)
                lidxb[pl.ds(g * 16, 16)] = jnp.where(ok, lv, NBSZ)
            cp1.wait()

            def ebody(e, carry2):
                v0 = eab[e, pl.ds(0, 16)]
                v1 = eab[e, pl.ds(16, 16)]
                contrib[e, pl.ds(0, 16)] = v0
                contrib[e, pl.ds(16, 16)] = v1
                contrib[e, pl.ds(32, 16)] = onehot32
                col = 32 * (e & 3)
                easb[e >> 2, pl.ds(col, 16)] = v0
                easb[e >> 2, pl.ds(col + 16, 16)] = v1
                return carry2

            lax.fori_loop(0, CH, ebody, 0)
            pltpu.sync_copy(easb, eas4_out.at[pl.ds(pl.multiple_of(e0 // 4, CH // 4), CH // 4)])
            pltpu.sync_copy(contrib, spacc.at[lidxb], add=True)
            return carry

        lax.fori_loop(0, nmine, chunk_body, 0)
        plsc.subcore_barrier()
        pltpu.sync_copy(
            spacc.at[pl.ds(pl.multiple_of(s * ROWS_PER_TILE, ROWS_PER_TILE), ROWS_PER_TILE)],
            degsum_out.at[pl.ds(pl.multiple_of(base + s * ROWS_PER_TILE, ROWS_PER_TILE), ROWS_PER_TILE)])
        plsc.subcore_barrier()


@functools.partial(
    pl.kernel,
    out_type=jax.ShapeDtypeStruct((NPAD, W128), jnp.float32),
    mesh=_mesh,
    scratch_types=[
        pltpu.VMEM_SHARED((NBSZ + 8, W128), jnp.float32),
        pltpu.VMEM((CH,), jnp.int32),
        pltpu.VMEM((CH,), jnp.int32),
        pltpu.VMEM((CH, W128), jnp.float32),
        pltpu.VMEM((CH, W128), jnp.float32),
        pltpu.VMEM((CH // 4, 256), jnp.float32),
        pltpu.VMEM((CH * 16,), jnp.float32),
        pltpu.VMEM((CH, W128), jnp.float32),
        pltpu.VMEM((CH,), jnp.int32),
        pltpu.VMEM((HID,), jnp.float32),
        pltpu.VMEM((16, 16), jnp.int32),
        pltpu.SemaphoreType.DMA,
        pltpu.SemaphoreType.DMA,
    ],
)
def _kedge(srcs_h, dsts_h, xlr_h, ee_h, accinit_h, bnds_h, att_h,
           accout_h,
           spacc, srcb, dstb, xlb, xrb, eeb, alphab, contrib, lidxb,
           attb, bndb, sem1, sem2):
    c = lax.axis_index("c")
    s = lax.axis_index("s")
    pltpu.sync_copy(att_h, attb)
    pltpu.sync_copy(bnds_h, bndb)
    iota = lax.iota(jnp.int32, 16)
    denmask = iota < HEADS
    zeros16 = jnp.zeros((16,), jnp.float32)
    attv = [attb[pl.ds(16 * h4, 16)] for h4 in range(HEADS)]

    def zbody(e, carry):
        for j in range(3):
            contrib[e, pl.ds(80 + 16 * j, 16)] = zeros16
        return carry
    lax.fori_loop(0, CH, zbody, 0)

    for bi in range(NBKT // 2):
        bkt = 2 * bi + c
        base = bkt * NBSZ
        bv = bndb[bkt]
        lo = bv[0]
        hi = bv[1]
        pltpu.sync_copy(
            accinit_h.at[pl.ds(pl.multiple_of(base + s * ROWS_PER_TILE, ROWS_PER_TILE), ROWS_PER_TILE)],
            spacc.at[pl.ds(pl.multiple_of(s * ROWS_PER_TILE, ROWS_PER_TILE), ROWS_PER_TILE)])
        plsc.subcore_barrier()
        nch = hi - lo
        nmine = jnp.maximum(0, (nch - s + 15) // 16)

        def chunk_body(i, carry):
            k = lo + s + 16 * i
            e0 = pl.multiple_of(k * CH, CH)
            pltpu.sync_copy(srcs_h.at[pl.ds(e0, CH)], srcb)
            pltpu.sync_copy(dsts_h.at[pl.ds(e0, CH)], dstb)
            g1 = pltpu.async_copy(xlr_h.at[srcb], xlb, sem1)
            g2 = pltpu.async_copy(xlr_h.at[dstb], xrb, sem2)
            pltpu.sync_copy(ee_h.at[pl.ds(pl.multiple_of(e0 // 4, CH // 4), CH // 4)], eeb)
            for g in range(CH // 16):
                dv = dstb[pl.ds(g * 16, 16)]
                lv = dv - base
                ok = (lv >= 0) & (lv < NBSZ)
                lidxb[pl.ds(g * 16, 16)] = jnp.where(ok, lv, NBSZ)
            g1.wait()
            g2.wait()

            def ebody(e, carry2):
                erow = e >> 2
                ecol = 64 * (e & 3)
                av = zeros16
                for h4 in range(HEADS):
                    m = (xlb[e, pl.ds(16 * h4, 16)]
                         + xrb[e, pl.ds(64 + 16 * h4, 16)]
                         + eeb[erow, pl.ds(ecol + 16 * h4, 16)])
                    m = jnp.where(m >= 0, m, 0.2 * m)
                    av = jnp.where(iota == h4, _vsum16(m * attv[h4], iota), av)
                alphab[pl.ds(16 * e, 16)] = av
                return carry2

            lax.fori_loop(0, CH, ebody, 0)

            def xbody(e, carry2):
                av = alphab[pl.ds(16 * e, 16)]
                alphab[pl.ds(16 * e, 16)] = jnp.exp(av)
                return carry2

            lax.fori_loop(0, CH, xbody, 0)

            def cbody(e, carry2):
                ev = alphab[pl.ds(16 * e, 16)]
                contrib[e, pl.ds(HID, 16)] = jnp.where(denmask, ev, 0.0)
                for h4 in range(HEADS):
                    sc_ = _splat(ev, h4)
                    contrib[e, pl.ds(16 * h4, 16)] = sc_ * xlb[e, pl.ds(16 * h4, 16)]
                return carry2

            lax.fori_loop(0, CH, cbody, 0)
            pltpu.sync_copy(contrib, spacc.at[lidxb], add=True)
            return carry

        lax.fori_loop(0, nmine, chunk_body, 0)
        plsc.subcore_barrier()
        pltpu.sync_copy(
            spacc.at[pl.ds(pl.multiple_of(s * ROWS_PER_TILE, ROWS_PER_TILE), ROWS_PER_TILE)],
            accout_h.at[pl.ds(pl.multiple_of(base + s * ROWS_PER_TILE, ROWS_PER_TILE), ROWS_PER_TILE)])
        plsc.subcore_barrier()


def kernel(x, edge_index, edge_attr, batch, global_attr, params):
    src = edge_index[0].astype(jnp.int32)
    dst = edge_index[1].astype(jnp.int32)

    eidx = jnp.arange(E, dtype=jnp.int32)
    dst_s, src_s, order = lax.sort([dst, src, eidx], num_keys=1)
    bounds = jnp.arange(0, NPAD + 1, NBSZ, dtype=jnp.int32)
    starts = jnp.searchsorted(dst_s, bounds).astype(jnp.int32)
    lo = starts[:NBKT] // CH
    hi = jnp.minimum((starts[1:] + CH - 1) // CH, NCHUNK)
    bnds = jnp.pad(jnp.stack([lo, hi], axis=1).astype(jnp.int32),
                   ((0, 16 - NBKT), (0, 14)))

    xp = jnp.pad(x, ((0, NPAD - N), (0, 0)))
    h = _encoder_pallas(xp, params["node_enc"], BLK)
    eap = _encoder_pallas(edge_attr, params["edge_enc"], EBLK, out_w=W128)

    zrows = jnp.zeros((NBSZ, W128), jnp.float32)
    ea_s4, degsum = _k0(dst_s, order, eap, zrows, bnds)

    for cp, nrm in zip(params["convs"], params["norms"]):
        att_flat = cp["att"].reshape(HID)
        xlr, accinit = _pre_pallas(h, degsum, cp, att_flat)
        ee = _ee_pallas(ea_s4, cp["W_e"])
        acc = _kedge(src_s, dst_s, xlr, ee, accinit, bnds, att_flat)
        h = _post_pallas(acc, cp["bias"], nrm, h)

    batch_pad = jnp.pad(batch.astype(jnp.float32), (0, NPAD - N),
                        constant_values=float(B))[:, None]
    sums, maxs = _pool_pallas(h, batch_pad)
    sums_t = jnp.sum(sums, axis=0)
    maxs_t = jnp.max(maxs, axis=0)
    cnt = (jnp.searchsorted(batch, jnp.arange(1, B + 1, dtype=batch.dtype))
           - jnp.searchsorted(batch, jnp.arange(0, B, dtype=batch.dtype))).astype(jnp.float32)
    out = _head_pallas(sums_t, maxs_t, cnt, global_attr, params["glob_enc"], params["head"])
    return out.reshape(B, 1)
